# baseline (speedup 1.0000x reference)
import jax
import jax.numpy as jnp
import numpy as np
from jax.experimental import pallas as pl
from jax.experimental.pallas import tpu as pltpu

_ANCHORS = np.array(
    [[16, 8], [23, 103], [28, 23], [56, 47], [96, 123], [157, 248]],
    dtype=np.float32,
)
_NUM_CLASSES = 80
_IMG_DIM = 512.0
_NA = 6
_ATTRS = 5 + _NUM_CLASSES
_NG = 64
_NPOS = _NG * _NG
_NCH = _NA * _ATTRS
_STRIDE = _IMG_DIM / _NG

_BB = 8
_YB = 4
_PB = _YB * _NG
_NYC = _NG // _YB


def _decode_kernel(x_ref, o_ref, stage_ref, sem):
    b8 = pl.program_id(0)
    yc = pl.program_id(1)
    step = b8 * _NYC + yc
    nsteps = pl.num_programs(0) * pl.num_programs(1)
    slot = step % 2

    def _wait_slot(s):
        for a in range(_NA):
            pltpu.make_async_copy(
                stage_ref.at[s, a],
                o_ref.at[:, pl.ds(0, _BB), pl.ds(0, _PB)],
                sem.at[s, a],
            ).wait()

    @pl.when(step >= 2)
    def _drain_two_ago():
        _wait_slot(slot)

    iota = jax.lax.broadcasted_iota(jnp.int32, (1, _PB), 1)
    gx = (iota % _NG).astype(jnp.float32)
    gy = (yc * _YB + iota // _NG).astype(jnp.float32)

    for bl in range(_BB):
        sub = x_ref[bl].reshape(_PB, _NCH)
        subT = sub.T
        for a in range(_NA):
            base = a * _ATTRS
            blk = subT[base:base + _ATTRS]
            r0 = (jax.nn.sigmoid(blk[0:1]) + gx) * _STRIDE
            r1 = (jax.nn.sigmoid(blk[1:2]) + gy) * _STRIDE
            r2 = jnp.exp(blk[2:3]) * float(_ANCHORS[a, 0])
            r3 = jnp.exp(blk[3:4]) * float(_ANCHORS[a, 1])
            r4 = jax.nn.sigmoid(blk[4:5])
            full = jnp.concatenate([r0, r1, r2, r3, r4, blk[5:]], axis=0)
            stage_ref[slot, a, :, bl, :] = full

    for a in range(_NA):
        pltpu.make_async_copy(
            stage_ref.at[slot, a],
            o_ref.at[
                :,
                pl.ds(b8 * _BB, _BB),
                pl.ds(a * _NPOS + yc * _PB, _PB),
            ],
            sem.at[slot, a],
        ).start()

    @pl.when(step == nsteps - 1)
    def _drain_tail():
        _wait_slot(1 - slot)
        _wait_slot(slot)


def kernel(x):
    nB = x.shape[0]
    xt = jnp.transpose(x, (0, 2, 3, 1))
    yt = pl.pallas_call(
        _decode_kernel,
        grid=(nB // _BB, _NYC),
        in_specs=[
            pl.BlockSpec((_BB, _YB, _NG, _NCH), lambda b8, yc: (b8, yc, 0, 0)),
        ],
        out_specs=pl.BlockSpec(memory_space=pltpu.MemorySpace.HBM),
        out_shape=jax.ShapeDtypeStruct((_ATTRS, nB, _NA * _NPOS), jnp.float32),
        scratch_shapes=[
            pltpu.VMEM((2, _NA, _ATTRS, _BB, _PB), jnp.float32),
            pltpu.SemaphoreType.DMA((2, _NA)),
        ],
        compiler_params=pltpu.CompilerParams(
            dimension_semantics=("arbitrary", "arbitrary"),
        ),
    )(xt)
    return jnp.transpose(yt, (1, 2, 0))


# speedup vs baseline: 1.0823x; 1.0823x over previous
import jax
import jax.numpy as jnp
import numpy as np
from jax.experimental import pallas as pl
from jax.experimental.pallas import tpu as pltpu

_ANCHORS = np.array(
    [[16, 8], [23, 103], [28, 23], [56, 47], [96, 123], [157, 248]],
    dtype=np.float32,
)
_NUM_CLASSES = 80
_IMG_DIM = 512.0
_NA = 6
_ATTRS = 5 + _NUM_CLASSES
_NG = 64
_NPOS = _NG * _NG
_NCH = _NA * _ATTRS
_STRIDE = _IMG_DIM / _NG

_BB = 8
_YB = 8
_PB = _YB * _NG
_NYC = _NG // _YB


def _decode_kernel(x_ref, o_ref, stage_ref, sem):
    b8 = pl.program_id(0)
    yc = pl.program_id(1)
    step = b8 * _NYC + yc
    nsteps = pl.num_programs(0) * pl.num_programs(1)
    slot = step % 2

    def _wait_slot(s):
        for a in range(_NA):
            pltpu.make_async_copy(
                stage_ref.at[s, a],
                o_ref.at[:, pl.ds(0, _BB), pl.ds(0, _PB)],
                sem.at[s, a],
            ).wait()

    @pl.when(step >= 2)
    def _drain_two_ago():
        _wait_slot(slot)

    iota = jax.lax.broadcasted_iota(jnp.int32, (1, _PB), 1)
    gx = (iota % _NG).astype(jnp.float32)
    gy = (yc * _YB + iota // _NG).astype(jnp.float32)

    for bl in range(_BB):
        sub = x_ref[bl].reshape(_PB, _NCH)
        subT = sub.T
        for a in range(_NA):
            base = a * _ATTRS
            blk = subT[base:base + _ATTRS]
            r0 = (jax.nn.sigmoid(blk[0:1]) + gx) * _STRIDE
            r1 = (jax.nn.sigmoid(blk[1:2]) + gy) * _STRIDE
            r2 = jnp.exp(blk[2:3]) * float(_ANCHORS[a, 0])
            r3 = jnp.exp(blk[3:4]) * float(_ANCHORS[a, 1])
            r4 = jax.nn.sigmoid(blk[4:5])
            full = jnp.concatenate([r0, r1, r2, r3, r4, blk[5:]], axis=0)
            stage_ref[slot, a, :, bl, :] = full

    for a in range(_NA):
        pltpu.make_async_copy(
            stage_ref.at[slot, a],
            o_ref.at[
                :,
                pl.ds(b8 * _BB, _BB),
                pl.ds(a * _NPOS + yc * _PB, _PB),
            ],
            sem.at[slot, a],
        ).start()

    @pl.when(step == nsteps - 1)
    def _drain_tail():
        _wait_slot(1 - slot)
        _wait_slot(slot)


def kernel(x):
    nB = x.shape[0]
    xt = jnp.transpose(x, (0, 2, 3, 1))
    yt = pl.pallas_call(
        _decode_kernel,
        grid=(nB // _BB, _NYC),
        in_specs=[
            pl.BlockSpec((_BB, _YB, _NG, _NCH), lambda b8, yc: (b8, yc, 0, 0)),
        ],
        out_specs=pl.BlockSpec(memory_space=pltpu.MemorySpace.HBM),
        out_shape=jax.ShapeDtypeStruct((_ATTRS, nB, _NA * _NPOS), jnp.float32),
        scratch_shapes=[
            pltpu.VMEM((2, _NA, _ATTRS, _BB, _PB), jnp.float32),
            pltpu.SemaphoreType.DMA((2, _NA)),
        ],
        compiler_params=pltpu.CompilerParams(
            dimension_semantics=("arbitrary", "arbitrary"),
        ),
    )(xt)
    return jnp.transpose(yt, (1, 2, 0))
